# baseline (speedup 1.0000x reference)
import dataclasses
import functools

import jax
import jax.numpy as jnp
from jax import lax
from jax.experimental import pallas as pl
from jax.experimental.pallas import tpu as pltpu
from jax.experimental.pallas import tpu_sc as plsc

_K_PROP = 10
_LAMBDA2 = 3.0
_GAMMA = 0.25
_BETA = 2.0

_N = 10000
_C = 8
_E = 320000
_NC = 2
_NS = 16
_NW = _NC * _NS
_NPW = 320
_NPAD = _NW * _NPW
_GPW = _NPW // 16
_ERW = 80
_EPAD = _NW * _ERW * 128
_SROWS = 8
_SPAD = _NS * _SROWS * 128
_NTRAIN = 5000


def _mlp_body(x_ref, w1_ref, b1_ref, wh_ref, bh_ref, wl_ref, bl_ref, h_ref):
    a = jnp.dot(x_ref[...], w1_ref[...], preferred_element_type=jnp.float32)
    a = jnp.maximum(a + b1_ref[...], 0.0)
    a = jnp.dot(a, wh_ref[...], preferred_element_type=jnp.float32)
    a = jnp.maximum(a + bh_ref[...], 0.0)
    h_ref[...] = jnp.dot(a, wl_ref[...], preferred_element_type=jnp.float32) + bl_ref[...]


def _mlp(xp, W1, b1, Wh, bh, Wl, bl):
    return pl.pallas_call(
        _mlp_body,
        out_shape=jax.ShapeDtypeStruct((_NPAD, _C), jnp.float32),
    )(xp, W1, b1.reshape(1, -1), Wh, bh.reshape(1, -1), Wl, bl.reshape(1, -1))


def _cidx(c):
    return jnp.full((16,), c, jnp.int32)


def _prop(h, srcp, dstp, sensp, sidxp):
    mesh = plsc.VectorSubcoreMesh(core_axis_name="c", subcore_axis_name="s",
                                  num_cores=_NC)
    cp = pltpu.CompilerParams()
    for fld, val in (("needs_layout_passes", False),
                     ("use_tc_tiling_on_sc", False)):
        if fld in pltpu.CompilerParams.__dataclass_fields__:
            cp = dataclasses.replace(cp, **{fld: val})

    @functools.partial(
        pl.kernel,
        out_type=(
            jax.ShapeDtypeStruct((_NPAD, _C), jnp.float32),
            jax.ShapeDtypeStruct((_NPAD, _C), jnp.float32),
            jax.ShapeDtypeStruct((_NC * _NPAD, _C), jnp.float32),
            jax.ShapeDtypeStruct((_NW * 16,), jnp.float32),
            jax.ShapeDtypeStruct((_NC * _NPAD,), jnp.float32),
            jax.ShapeDtypeStruct((_NPAD,), jnp.float32),
        ),
        mesh=mesh,
        compiler_params=cp,
        scratch_types=[
            pltpu.VMEM_SHARED((_NPAD, _C), jnp.float32),
            pltpu.VMEM_SHARED((_NPAD, _C), jnp.float32),
            pltpu.VMEM_SHARED((_NPAD,), jnp.float32),
            pltpu.VMEM_SHARED((_NPAD,), jnp.float32),
            pltpu.VMEM_SHARED((_NS * 16,), jnp.float32),
            pltpu.VMEM((_ERW, 128), jnp.int32),
            pltpu.VMEM((_ERW, 128), jnp.int32),
            pltpu.VMEM((8, 128, _C), jnp.float32),
            pltpu.VMEM((_C, _NPW), jnp.float32),
            pltpu.VMEM((_C, _NPW), jnp.float32),
            pltpu.VMEM((_C, _NPW), jnp.float32),
            pltpu.VMEM((_NPW, _C), jnp.float32),
            pltpu.VMEM((_NPW, _C), jnp.float32),
            pltpu.VMEM((2 * _NPW, _C), jnp.float32),
            pltpu.VMEM((2 * _NPW, _C), jnp.float32),
            pltpu.VMEM((2 * _NPW,), jnp.float32),
            pltpu.VMEM((2 * _NPW,), jnp.float32),
            pltpu.VMEM((_NPW,), jnp.float32),
            pltpu.VMEM((_NPW,), jnp.float32),
            pltpu.VMEM((_NPW,), jnp.float32),
            pltpu.VMEM((_NPW,), jnp.float32),
            pltpu.VMEM((_SROWS, 128), jnp.int32),
            pltpu.VMEM((_SROWS, 128), jnp.int32),
            pltpu.VMEM((_SROWS, 128), jnp.float32),
            pltpu.VMEM((128,), jnp.float32),
            pltpu.VMEM((_NS * 16,), jnp.float32),
            pltpu.VMEM((_NW * 16,), jnp.float32),
            pltpu.VMEM((16,), jnp.float32),
            pltpu.SemaphoreType.REGULAR,
        ] + [pltpu.SemaphoreType.DMA] * 16,
    )
    def k(h_hbm, src_hbm, dst_hbm, sens_hbm, sidx_hbm,
          out_hbm, zf_hbm, agg_hbm, red_hbm, deg_hbm, delta_hbm,
          zf_sh, agg_sh, deg_sh, delta_sh, red_sh,
          src_v, dst_v, gbufs_r, h_pl, z_pl, p_pl, rows_a, rows_b,
          zfstage, zero2d, stage640, zero1d, dinv_v, delta_v, dega_v,
          degb_v, sidx_v, sval_v, vals_v, ones_v, red_v, redg_v, vec16,
          xsem, *sems):
        gbufs = tuple(gbufs_r.at[b] for b in range(8))
        gsems = sems[:8]
        ssems = sems[8:]
        cid = lax.axis_index("c")
        sid = lax.axis_index("s")
        wid = cid * _NS + sid
        nbase = wid * _NPW
        sbase = sid * 2 * _NPW
        iota = lax.iota(jnp.int32, 16)

        def gbar():
            plsc.subcore_barrier()
            pl.semaphore_signal(xsem, 1, core_index=1 - cid)
            pl.semaphore_wait(xsem, 1)

        pltpu.sync_copy(src_hbm.at[pl.ds(wid * _ERW, _ERW)], src_v)
        pltpu.sync_copy(dst_hbm.at[pl.ds(wid * _ERW, _ERW)], dst_v)

        @pl.loop(0, 2 * _GPW)
        def _(g):
            zv = jnp.zeros((16,), jnp.float32)
            zero1d[pl.ds(g * 16, 16)] = zv
            ridx = g * 16 + iota
            for c in range(_C):
                plsc.store_scatter(zero2d, [ridx, _cidx(c)], zv)

        @pl.loop(0, 8)
        def _(g):
            ones_v[pl.ds(g * 16, 16)] = jnp.ones((16,), jnp.float32)

        pltpu.sync_copy(zero1d, deg_sh.at[pl.ds(sbase, 2 * _NPW)])
        pltpu.sync_copy(zero1d, delta_sh.at[pl.ds(sbase, 2 * _NPW)])
        plsc.subcore_barrier()

        @pl.loop(0, _ERW)
        def _(j):
            pltpu.sync_copy(ones_v, deg_sh.at[dst_v.at[j]], add=True)

        @pl.when(cid == 0)
        def _():
            pltpu.sync_copy(sidx_hbm.at[pl.ds(sid * _SROWS, _SROWS)], sidx_v)
            for r in range(_SROWS):
                pltpu.sync_copy(sens_hbm.at[sidx_v.at[r]], sval_v.at[r])
            acc = jnp.zeros((16,), jnp.float32)
            for r in range(_SROWS):
                for g in range(8):
                    acc = acc + sval_v[r, pl.ds(g * 16, 16)].astype(jnp.float32)
            vec16[...] = jnp.where(iota == 0, jnp.sum(acc), 0.0)
            pltpu.sync_copy(vec16, red_sh.at[pl.ds(sid * 16, 16)])
            plsc.subcore_barrier()
            pltpu.sync_copy(red_sh, red_v)
            tot = red_v[pl.ds(0, 16)]
            for i in range(1, _NS):
                tot = tot + red_v[pl.ds(i * 16, 16)]
            n1b = jnp.full((16,), tot[0])
            r1v = 1.0 / jnp.maximum(n1b, 1.0)
            r0v = -1.0 / jnp.maximum(float(_NTRAIN) - n1b, 1.0)
            for r in range(_SROWS):
                for g in range(8):
                    sl = pl.ds(g * 16, 16)
                    sv = sval_v[r, sl]
                    ix = sidx_v[r, sl]
                    val = jnp.where(sv > 0, r1v, r0v)
                    vals_v[r, sl] = jnp.where(ix < _N, val, 0.0)
            for r in range(_SROWS):
                pltpu.sync_copy(vals_v.at[r], delta_sh.at[sidx_v.at[r]],
                                add=True)
            plsc.subcore_barrier()
            pltpu.sync_copy(delta_sh.at[pl.ds(sbase, 2 * _NPW)], stage640)
            pltpu.sync_copy(stage640, delta_hbm.at[pl.ds(sbase, 2 * _NPW)])

        plsc.subcore_barrier()
        pltpu.sync_copy(deg_sh.at[pl.ds(sbase, 2 * _NPW)], stage640)
        pltpu.sync_copy(stage640,
                        deg_hbm.at[pl.ds(cid * _NPAD + sbase, 2 * _NPW)])
        gbar()

        pltpu.sync_copy(deg_hbm.at[pl.ds(nbase, _NPW)], dega_v)
        pltpu.sync_copy(deg_hbm.at[pl.ds(_NPAD + nbase, _NPW)], degb_v)
        pltpu.sync_copy(delta_hbm.at[pl.ds(nbase, _NPW)], delta_v)
        pltpu.sync_copy(h_hbm.at[pl.ds(nbase, _NPW)], rows_a)

        @pl.loop(0, _GPW)
        def _(g):
            sl = pl.ds(g * 16, 16)
            d = jnp.maximum(dega_v[sl] + degb_v[sl], 1.0)
            db = plsc.bitcast(d, jnp.int32)
            y = plsc.bitcast(jnp.int32(0x5F3759DF) - (db >> 1), jnp.float32)
            for _i in range(3):
                y = y * (1.5 - 0.5 * d * y * y)
            dinv_v[sl] = y
            ridx = g * 16 + iota
            for c in range(_C):
                hv = plsc.load_gather(rows_a, [ridx, _cidx(c)])
                h_pl[c, sl] = hv
                z_pl[c, sl] = hv

        def step(_k, u):
            def ga(g, accs):
                sl = pl.ds(g * 16, 16)
                zs = [z_pl[c, sl] for c in range(_C)]
                m = zs[0]
                for c in range(1, _C):
                    m = jnp.maximum(m, zs[c])
                es = [jnp.exp(zs[c] - m) for c in range(_C)]
                s = es[0]
                for c in range(1, _C):
                    s = s + es[c]
                rinv = 1.0 / s
                dv = delta_v[sl]
                out = []
                for c in range(_C):
                    p = es[c] * rinv
                    p_pl[c, sl] = p
                    out.append(accs[c] + dv * p)
                return tuple(out)

            accs = lax.fori_loop(
                0, _GPW, ga,
                tuple(jnp.zeros((16,), jnp.float32) for _ in range(_C)))
            ap = jnp.zeros((16,), jnp.float32)
            for c in range(_C):
                ap = jnp.where(iota == c, jnp.sum(accs[c]), ap)
            vec16[...] = ap
            pltpu.sync_copy(vec16, red_hbm.at[pl.ds(wid * 16, 16)])
            gbar()
            pltpu.sync_copy(red_hbm, redg_v)
            a = redg_v[pl.ds(0, 16)]
            for i in range(1, _NW):
                a = a + redg_v[pl.ds(i * 16, 16)]
            u = jnp.clip(u + _BETA * a, -_LAMBDA2, _LAMBDA2)
            us = [jnp.full((16,), u[c]) for c in range(_C)]

            def gb(g, carry):
                sl = pl.ds(g * 16, 16)
                ps = [p_pl[c, sl] for c in range(_C)]
                pu = ps[0] * us[0]
                for c in range(1, _C):
                    pu = pu + ps[c] * us[c]
                dv = delta_v[sl] * _GAMMA
                divv = dinv_v[sl]
                ridx = g * 16 + iota
                for c in range(_C):
                    zf = (z_pl[c, sl] - dv * (ps[c] * (us[c] - pu))) * divv
                    plsc.store_scatter(rows_a, [ridx, _cidx(c)], zf)
                return carry

            lax.fori_loop(0, _GPW, gb, 0)
            pltpu.sync_copy(rows_a, zf_hbm.at[pl.ds(nbase, _NPW)])
            gbar()

            pltpu.sync_copy(zf_hbm.at[pl.ds(sbase, 2 * _NPW)],
                            zf_sh.at[pl.ds(sbase, 2 * _NPW)])
            pltpu.sync_copy(zero2d, agg_sh.at[pl.ds(sbase, 2 * _NPW)])
            plsc.subcore_barrier()

            def _gath(j, b):
                return pltpu.make_async_copy(
                    zf_sh.at[src_v.at[j]], gbufs[b], gsems[b])

            def _scat(j, b):
                return pltpu.make_async_copy(
                    gbufs[b], agg_sh.at[dst_v.at[j]], ssems[b])

            for b0 in range(4):
                _gath(b0, b0).start()
            for j0 in range(4):
                _gath(j0, j0).wait()
                _scat(j0, j0).start(add=True)
                _gath(j0 + 4, j0 + 4).start()

            @pl.loop(0, (_ERW - 8) // 8)
            def _(i):
                j = 8 * i + 4
                for t in range(8):
                    b = (4 + t) % 8
                    _gath(j + t, b).wait()
                    _scat(j + t, b).start(add=True)
                    b2 = (b + 4) % 8
                    _scat(j + t - 4, b2).wait()
                    _gath(j + t + 4, b2).start()

            for t in range(4):
                j0, b0 = _ERW - 4 + t, (4 + t) % 8
                _gath(j0, b0).wait()
                _scat(j0, b0).start(add=True)
                _scat(j0 - 4, (b0 + 4) % 8).wait()
            for t in range(4):
                _scat(_ERW - 4 + t, (4 + t) % 8).wait()

            plsc.subcore_barrier()

            pltpu.sync_copy(
                agg_sh.at[pl.ds(sbase, 2 * _NPW)],
                agg_hbm.at[pl.ds(cid * _NPAD + sbase, 2 * _NPW)])
            gbar()

            pltpu.sync_copy(
                agg_hbm.at[pl.ds((1 - cid) * _NPAD + nbase, _NPW)], rows_a)
            pltpu.sync_copy(agg_sh.at[pl.ds(nbase, _NPW)], rows_b)

            def gc(g, carry):
                sl = pl.ds(g * 16, 16)
                divv = dinv_v[sl]
                ridx = g * 16 + iota
                for c in range(_C):
                    av = (plsc.load_gather(rows_a, [ridx, _cidx(c)])
                          + plsc.load_gather(rows_b, [ridx, _cidx(c)]))
                    z_pl[c, sl] = (_GAMMA * h_pl[c, sl]
                                   + (1.0 - _GAMMA) * divv * av)
                return carry

            lax.fori_loop(0, _GPW, gc, 0)
            return u

        lax.fori_loop(0, _K_PROP, step, jnp.zeros((16,), jnp.float32))

        def go(g, carry):
            sl = pl.ds(g * 16, 16)
            ridx = g * 16 + iota
            for c in range(_C):
                plsc.store_scatter(rows_a, [ridx, _cidx(c)], z_pl[c, sl])
            return carry

        lax.fori_loop(0, _GPW, go, 0)
        pltpu.sync_copy(rows_a, out_hbm.at[pl.ds(nbase, _NPW)])

    return k(h, srcp, dstp, sensp, sidxp)[0]


def kernel(x, edge_index, sensitive_attr, idx_sens_train, W1, b1, Wh, bh, Wl, bl):
    xp = jnp.pad(x, ((0, _NPAD - _N), (0, 0)))
    h = _mlp(xp, W1, b1, Wh, bh, Wl, bl)

    n_epad = _EPAD - _E
    pad_idx = (jnp.arange(n_epad, dtype=jnp.int32) % (_NPAD - _N)) + _N
    srcp = jnp.concatenate([edge_index[0], pad_idx]).reshape(_EPAD // 128, 128)
    dstp = jnp.concatenate([edge_index[1], pad_idx]).reshape(_EPAD // 128, 128)
    sensp = jnp.pad(sensitive_attr, (0, _NPAD - _N))
    sidxp = jnp.concatenate(
        [idx_sens_train,
         jnp.full((_SPAD - _NTRAIN,), _N, jnp.int32)]).reshape(_SPAD // 128, 128)

    z = _prop(h, srcp, dstp, sensp, sidxp)
    return z[:_N]


# speedup vs baseline: 1.0170x; 1.0170x over previous
import dataclasses
import functools

import jax
import jax.numpy as jnp
from jax import lax
from jax.experimental import pallas as pl
from jax.experimental.pallas import tpu as pltpu
from jax.experimental.pallas import tpu_sc as plsc

_K_PROP = 10
_LAMBDA2 = 3.0
_GAMMA = 0.25
_BETA = 2.0

_N = 10000
_C = 8
_E = 320000
_NC = 2
_NS = 16
_NW = _NC * _NS
_NPW = 320
_NPAD = _NW * _NPW
_GPW = _NPW // 16
_ERW = 80
_EPAD = _NW * _ERW * 128
_SROWS = 8
_SPAD = _NS * _SROWS * 128
_NTRAIN = 5000


def _mlp_body(x_ref, w1_ref, b1_ref, wh_ref, bh_ref, wl_ref, bl_ref, h_ref):
    a = jnp.dot(x_ref[...], w1_ref[...], preferred_element_type=jnp.float32)
    a = jnp.maximum(a + b1_ref[...], 0.0)
    a = jnp.dot(a, wh_ref[...], preferred_element_type=jnp.float32)
    a = jnp.maximum(a + bh_ref[...], 0.0)
    h_ref[...] = jnp.dot(a, wl_ref[...], preferred_element_type=jnp.float32) + bl_ref[...]


def _mlp(xp, W1, b1, Wh, bh, Wl, bl):
    return pl.pallas_call(
        _mlp_body,
        out_shape=jax.ShapeDtypeStruct((_NPAD, _C), jnp.float32),
    )(xp, W1, b1.reshape(1, -1), Wh, bh.reshape(1, -1), Wl, bl.reshape(1, -1))


def _cidx(c):
    return jnp.full((16,), c, jnp.int32)


def _prop(h, srcp, dstp, sensp, sidxp):
    mesh = plsc.VectorSubcoreMesh(core_axis_name="c", subcore_axis_name="s",
                                  num_cores=_NC)
    cp = pltpu.CompilerParams()
    for fld, val in (("needs_layout_passes", False),
                     ("use_tc_tiling_on_sc", False)):
        if fld in pltpu.CompilerParams.__dataclass_fields__:
            cp = dataclasses.replace(cp, **{fld: val})

    @functools.partial(
        pl.kernel,
        out_type=(
            jax.ShapeDtypeStruct((_NPAD, _C), jnp.float32),
            jax.ShapeDtypeStruct((_NPAD, _C), jnp.float32),
            jax.ShapeDtypeStruct((_NC * _NPAD, _C), jnp.float32),
            jax.ShapeDtypeStruct((_NW * 16,), jnp.float32),
            jax.ShapeDtypeStruct((_NC * _NPAD,), jnp.float32),
            jax.ShapeDtypeStruct((_NPAD,), jnp.float32),
        ),
        mesh=mesh,
        compiler_params=cp,
        scratch_types=[
            pltpu.VMEM_SHARED((_NPAD, _C), jnp.float32),
            pltpu.VMEM_SHARED((_NPAD, _C), jnp.float32),
            pltpu.VMEM_SHARED((_NPAD,), jnp.float32),
            pltpu.VMEM_SHARED((_NPAD,), jnp.float32),
            pltpu.VMEM_SHARED((_NS * 16,), jnp.float32),
            pltpu.VMEM((_ERW, 128), jnp.int32),
            pltpu.VMEM((_ERW, 128), jnp.int32),
            pltpu.VMEM((8, 128, _C), jnp.float32),
            pltpu.VMEM((_C, _NPW), jnp.float32),
            pltpu.VMEM((_C, _NPW), jnp.float32),
            pltpu.VMEM((_C, _NPW), jnp.float32),
            pltpu.VMEM((_NPW, _C), jnp.float32),
            pltpu.VMEM((_NPW, _C), jnp.float32),
            pltpu.VMEM((2 * _NPW, _C), jnp.float32),
            pltpu.VMEM((2 * _NPW, _C), jnp.float32),
            pltpu.VMEM((2 * _NPW,), jnp.float32),
            pltpu.VMEM((2 * _NPW,), jnp.float32),
            pltpu.VMEM((_NPW,), jnp.float32),
            pltpu.VMEM((_NPW,), jnp.float32),
            pltpu.VMEM((_NPW,), jnp.float32),
            pltpu.VMEM((_NPW,), jnp.float32),
            pltpu.VMEM((_SROWS, 128), jnp.int32),
            pltpu.VMEM((_SROWS, 128), jnp.int32),
            pltpu.VMEM((_SROWS, 128), jnp.float32),
            pltpu.VMEM((128,), jnp.float32),
            pltpu.VMEM((_NS * 16,), jnp.float32),
            pltpu.VMEM((_NW * 16,), jnp.float32),
            pltpu.VMEM((16,), jnp.float32),
            pltpu.SemaphoreType.REGULAR,
        ] + [pltpu.SemaphoreType.DMA] * 17,
    )
    def k(h_hbm, src_hbm, dst_hbm, sens_hbm, sidx_hbm,
          out_hbm, zf_hbm, agg_hbm, red_hbm, deg_hbm, delta_hbm,
          zf_sh, agg_sh, deg_sh, delta_sh, red_sh,
          src_v, dst_v, gbufs_r, h_pl, z_pl, p_pl, rows_a, rows_b,
          zfstage, zero2d, stage640, zero1d, dinv_v, delta_v, dega_v,
          degb_v, sidx_v, sval_v, vals_v, ones_v, red_v, redg_v, vec16,
          xsem, *sems):
        gbufs = tuple(gbufs_r.at[b] for b in range(8))
        gsems = sems[:8]
        ssems = sems[8:]
        cid = lax.axis_index("c")
        sid = lax.axis_index("s")
        wid = cid * _NS + sid
        nbase = wid * _NPW
        sbase = sid * 2 * _NPW
        iota = lax.iota(jnp.int32, 16)

        def gbar():
            plsc.subcore_barrier()
            pl.semaphore_signal(xsem, 1, core_index=1 - cid)
            pl.semaphore_wait(xsem, 1)

        pltpu.sync_copy(src_hbm.at[pl.ds(wid * _ERW, _ERW)], src_v)
        pltpu.sync_copy(dst_hbm.at[pl.ds(wid * _ERW, _ERW)], dst_v)

        @pl.loop(0, 2 * _GPW)
        def _(g):
            zv = jnp.zeros((16,), jnp.float32)
            zero1d[pl.ds(g * 16, 16)] = zv
            ridx = g * 16 + iota
            for c in range(_C):
                plsc.store_scatter(zero2d, [ridx, _cidx(c)], zv)

        @pl.loop(0, 8)
        def _(g):
            ones_v[pl.ds(g * 16, 16)] = jnp.ones((16,), jnp.float32)

        pltpu.sync_copy(zero1d, deg_sh.at[pl.ds(sbase, 2 * _NPW)])
        pltpu.sync_copy(zero1d, delta_sh.at[pl.ds(sbase, 2 * _NPW)])
        plsc.subcore_barrier()

        @pl.loop(0, _ERW)
        def _(j):
            pltpu.sync_copy(ones_v, deg_sh.at[dst_v.at[j]], add=True)

        @pl.when(cid == 0)
        def _():
            pltpu.sync_copy(sidx_hbm.at[pl.ds(sid * _SROWS, _SROWS)], sidx_v)
            for r in range(_SROWS):
                pltpu.sync_copy(sens_hbm.at[sidx_v.at[r]], sval_v.at[r])
            acc = jnp.zeros((16,), jnp.float32)
            for r in range(_SROWS):
                for g in range(8):
                    acc = acc + sval_v[r, pl.ds(g * 16, 16)].astype(jnp.float32)
            vec16[...] = jnp.where(iota == 0, jnp.sum(acc), 0.0)
            pltpu.sync_copy(vec16, red_sh.at[pl.ds(sid * 16, 16)])
            plsc.subcore_barrier()
            pltpu.sync_copy(red_sh, red_v)
            tot = red_v[pl.ds(0, 16)]
            for i in range(1, _NS):
                tot = tot + red_v[pl.ds(i * 16, 16)]
            n1b = jnp.full((16,), tot[0])
            r1v = 1.0 / jnp.maximum(n1b, 1.0)
            r0v = -1.0 / jnp.maximum(float(_NTRAIN) - n1b, 1.0)
            for r in range(_SROWS):
                for g in range(8):
                    sl = pl.ds(g * 16, 16)
                    sv = sval_v[r, sl]
                    ix = sidx_v[r, sl]
                    val = jnp.where(sv > 0, r1v, r0v)
                    vals_v[r, sl] = jnp.where(ix < _N, val, 0.0)
            for r in range(_SROWS):
                pltpu.sync_copy(vals_v.at[r], delta_sh.at[sidx_v.at[r]],
                                add=True)
            plsc.subcore_barrier()
            pltpu.sync_copy(delta_sh.at[pl.ds(sbase, 2 * _NPW)], stage640)
            pltpu.sync_copy(stage640, delta_hbm.at[pl.ds(sbase, 2 * _NPW)])

        plsc.subcore_barrier()
        pltpu.sync_copy(deg_sh.at[pl.ds(sbase, 2 * _NPW)], stage640)
        pltpu.sync_copy(stage640,
                        deg_hbm.at[pl.ds(cid * _NPAD + sbase, 2 * _NPW)])
        gbar()

        pltpu.sync_copy(deg_hbm.at[pl.ds(nbase, _NPW)], dega_v)
        pltpu.sync_copy(deg_hbm.at[pl.ds(_NPAD + nbase, _NPW)], degb_v)
        pltpu.sync_copy(delta_hbm.at[pl.ds(nbase, _NPW)], delta_v)
        pltpu.sync_copy(h_hbm.at[pl.ds(nbase, _NPW)], rows_a)

        @pl.loop(0, _GPW)
        def _(g):
            sl = pl.ds(g * 16, 16)
            d = jnp.maximum(dega_v[sl] + degb_v[sl], 1.0)
            db = plsc.bitcast(d, jnp.int32)
            y = plsc.bitcast(jnp.int32(0x5F3759DF) - (db >> 1), jnp.float32)
            for _i in range(3):
                y = y * (1.5 - 0.5 * d * y * y)
            dinv_v[sl] = y
            ridx = g * 16 + iota
            for c in range(_C):
                hv = plsc.load_gather(rows_a, [ridx, _cidx(c)])
                h_pl[c, sl] = hv
                z_pl[c, sl] = hv

        def step(_k, u):
            def ga(g, accs):
                sl = pl.ds(g * 16, 16)
                zs = [z_pl[c, sl] for c in range(_C)]
                m = zs[0]
                for c in range(1, _C):
                    m = jnp.maximum(m, zs[c])
                es = [jnp.exp(zs[c] - m) for c in range(_C)]
                s = es[0]
                for c in range(1, _C):
                    s = s + es[c]
                rinv = 1.0 / s
                dv = delta_v[sl]
                out = []
                for c in range(_C):
                    p = es[c] * rinv
                    p_pl[c, sl] = p
                    out.append(accs[c] + dv * p)
                return tuple(out)

            accs = lax.fori_loop(
                0, _GPW, ga,
                tuple(jnp.zeros((16,), jnp.float32) for _ in range(_C)))
            ap = jnp.zeros((16,), jnp.float32)
            for c in range(_C):
                ap = jnp.where(iota == c, jnp.sum(accs[c]), ap)
            vec16[...] = ap
            pltpu.sync_copy(vec16, red_hbm.at[pl.ds(wid * 16, 16)])
            gbar()
            pltpu.make_async_copy(zero2d, agg_sh.at[pl.ds(sbase, 2 * _NPW)],
                                  sems[16]).start()
            pltpu.sync_copy(red_hbm, redg_v)
            a = redg_v[pl.ds(0, 16)]
            for i in range(1, _NW):
                a = a + redg_v[pl.ds(i * 16, 16)]
            u = jnp.clip(u + _BETA * a, -_LAMBDA2, _LAMBDA2)
            us = [jnp.full((16,), u[c]) for c in range(_C)]

            def gb(g, carry):
                sl = pl.ds(g * 16, 16)
                ps = [p_pl[c, sl] for c in range(_C)]
                pu = ps[0] * us[0]
                for c in range(1, _C):
                    pu = pu + ps[c] * us[c]
                dv = delta_v[sl] * _GAMMA
                divv = dinv_v[sl]
                ridx = g * 16 + iota
                for c in range(_C):
                    zf = (z_pl[c, sl] - dv * (ps[c] * (us[c] - pu))) * divv
                    plsc.store_scatter(rows_a, [ridx, _cidx(c)], zf)
                return carry

            lax.fori_loop(0, _GPW, gb, 0)
            pltpu.sync_copy(rows_a, zf_hbm.at[pl.ds(nbase, _NPW)])
            pltpu.sync_copy(rows_a, zf_sh.at[pl.ds(nbase, _NPW)])
            gbar()

            pbase = (1 - cid) * (_NS * _NPW) + sid * _NPW
            pltpu.sync_copy(zf_hbm.at[pl.ds(pbase, _NPW)], rows_b)
            pltpu.sync_copy(rows_b, zf_sh.at[pl.ds(pbase, _NPW)])
            pltpu.make_async_copy(zero2d, agg_sh.at[pl.ds(sbase, 2 * _NPW)],
                                  sems[16]).wait()
            plsc.subcore_barrier()

            def _gath(j, b):
                return pltpu.make_async_copy(
                    zf_sh.at[src_v.at[j]], gbufs[b], gsems[b])

            def _scat(j, b):
                return pltpu.make_async_copy(
                    gbufs[b], agg_sh.at[dst_v.at[j]], ssems[b])

            for b0 in range(4):
                _gath(b0, b0).start()
            for j0 in range(4):
                _gath(j0, j0).wait()
                _scat(j0, j0).start(add=True)
                _gath(j0 + 4, j0 + 4).start()

            @pl.loop(0, (_ERW - 8) // 8)
            def _(i):
                j = 8 * i + 4
                for t in range(8):
                    b = (4 + t) % 8
                    _gath(j + t, b).wait()
                    _scat(j + t, b).start(add=True)
                    b2 = (b + 4) % 8
                    _scat(j + t - 4, b2).wait()
                    _gath(j + t + 4, b2).start()

            for t in range(4):
                j0, b0 = _ERW - 4 + t, (4 + t) % 8
                _gath(j0, b0).wait()
                _scat(j0, b0).start(add=True)
                _scat(j0 - 4, (b0 + 4) % 8).wait()
            for t in range(4):
                _scat(_ERW - 4 + t, (4 + t) % 8).wait()

            plsc.subcore_barrier()

            pltpu.sync_copy(agg_sh.at[pl.ds(sbase, 2 * _NPW)], zfstage)
            pltpu.sync_copy(
                zfstage, agg_hbm.at[pl.ds(cid * _NPAD + sbase, 2 * _NPW)])
            gbar()

            pltpu.sync_copy(
                agg_hbm.at[pl.ds((1 - cid) * _NPAD + nbase, _NPW)], rows_a)
            pltpu.sync_copy(agg_sh.at[pl.ds(nbase, _NPW)], rows_b)

            def gc(g, carry):
                sl = pl.ds(g * 16, 16)
                divv = dinv_v[sl]
                ridx = g * 16 + iota
                for c in range(_C):
                    av = (plsc.load_gather(rows_a, [ridx, _cidx(c)])
                          + plsc.load_gather(rows_b, [ridx, _cidx(c)]))
                    z_pl[c, sl] = (_GAMMA * h_pl[c, sl]
                                   + (1.0 - _GAMMA) * divv * av)
                return carry

            lax.fori_loop(0, _GPW, gc, 0)
            return u

        lax.fori_loop(0, _K_PROP, step, jnp.zeros((16,), jnp.float32))

        def go(g, carry):
            sl = pl.ds(g * 16, 16)
            ridx = g * 16 + iota
            for c in range(_C):
                plsc.store_scatter(rows_a, [ridx, _cidx(c)], z_pl[c, sl])
            return carry

        lax.fori_loop(0, _GPW, go, 0)
        pltpu.sync_copy(rows_a, out_hbm.at[pl.ds(nbase, _NPW)])

    return k(h, srcp, dstp, sensp, sidxp)[0]


def kernel(x, edge_index, sensitive_attr, idx_sens_train, W1, b1, Wh, bh, Wl, bl):
    xp = jnp.pad(x, ((0, _NPAD - _N), (0, 0)))
    h = _mlp(xp, W1, b1, Wh, bh, Wl, bl)

    n_epad = _EPAD - _E
    pad_idx = (jnp.arange(n_epad, dtype=jnp.int32) % (_NPAD - _N)) + _N
    srcp = jnp.concatenate([edge_index[0], pad_idx]).reshape(_EPAD // 128, 128)
    dstp = jnp.concatenate([edge_index[1], pad_idx]).reshape(_EPAD // 128, 128)
    sensp = jnp.pad(sensitive_attr, (0, _NPAD - _N))
    sidxp = jnp.concatenate(
        [idx_sens_train,
         jnp.full((_SPAD - _NTRAIN,), _N, jnp.int32)]).reshape(_SPAD // 128, 128)

    z = _prop(h, srcp, dstp, sensp, sidxp)
    return z[:_N]


# speedup vs baseline: 1.0300x; 1.0128x over previous
import dataclasses
import functools

import jax
import jax.numpy as jnp
from jax import lax
from jax.experimental import pallas as pl
from jax.experimental.pallas import tpu as pltpu
from jax.experimental.pallas import tpu_sc as plsc

_K_PROP = 10
_LAMBDA2 = 3.0
_GAMMA = 0.25
_BETA = 2.0

_N = 10000
_C = 8
_E = 320000
_NC = 2
_NS = 16
_NW = _NC * _NS
_NPW = 320
_NPAD = _NW * _NPW
_GPW = _NPW // 16
_ERW = 80
_EPAD = _NW * _ERW * 128
_SROWS = 8
_SPAD = _NS * _SROWS * 128
_NTRAIN = 5000


def _mlp_body(x_ref, w1_ref, b1_ref, wh_ref, bh_ref, wl_ref, bl_ref, h_ref):
    a = jnp.dot(x_ref[...], w1_ref[...], preferred_element_type=jnp.float32)
    a = jnp.maximum(a + b1_ref[...], 0.0)
    a = jnp.dot(a, wh_ref[...], preferred_element_type=jnp.float32)
    a = jnp.maximum(a + bh_ref[...], 0.0)
    h_ref[...] = jnp.dot(a, wl_ref[...], preferred_element_type=jnp.float32) + bl_ref[...]


def _mlp(xp, W1, b1, Wh, bh, Wl, bl):
    return pl.pallas_call(
        _mlp_body,
        out_shape=jax.ShapeDtypeStruct((_NPAD, _C), jnp.float32),
    )(xp, W1, b1.reshape(1, -1), Wh, bh.reshape(1, -1), Wl, bl.reshape(1, -1))


def _cidx(c):
    return jnp.full((16,), c, jnp.int32)


def _prop(h, srcp, dstp, sensp, sidxp):
    mesh = plsc.VectorSubcoreMesh(core_axis_name="c", subcore_axis_name="s",
                                  num_cores=_NC)
    cp = pltpu.CompilerParams()
    for fld, val in (("needs_layout_passes", False),
                     ("use_tc_tiling_on_sc", False)):
        if fld in pltpu.CompilerParams.__dataclass_fields__:
            cp = dataclasses.replace(cp, **{fld: val})

    @functools.partial(
        pl.kernel,
        out_type=(
            jax.ShapeDtypeStruct((_NPAD, _C), jnp.float32),
            jax.ShapeDtypeStruct((_NPAD, _C), jnp.float32),
            jax.ShapeDtypeStruct((_NC * _NPAD, _C), jnp.float32),
            jax.ShapeDtypeStruct((_NW * 16,), jnp.float32),
            jax.ShapeDtypeStruct((_NC * _NPAD,), jnp.float32),
            jax.ShapeDtypeStruct((_NPAD,), jnp.float32),
        ),
        mesh=mesh,
        compiler_params=cp,
        scratch_types=[
            pltpu.VMEM_SHARED((_NPAD, _C), jnp.float32),
            pltpu.VMEM_SHARED((_NPAD, _C), jnp.float32),
            pltpu.VMEM_SHARED((_NPAD,), jnp.float32),
            pltpu.VMEM_SHARED((_NPAD,), jnp.float32),
            pltpu.VMEM_SHARED((_NS * 16,), jnp.float32),
            pltpu.VMEM((_ERW, 128), jnp.int32),
            pltpu.VMEM((_ERW, 128), jnp.int32),
            pltpu.VMEM((8, 128, _C), jnp.float32),
            pltpu.VMEM((_C, _NPW), jnp.float32),
            pltpu.VMEM((_C, _NPW), jnp.float32),
            pltpu.VMEM((_C, _NPW), jnp.float32),
            pltpu.VMEM((_NPW, _C), jnp.float32),
            pltpu.VMEM((_NPW, _C), jnp.float32),
            pltpu.VMEM((2 * _NPW, _C), jnp.float32),
            pltpu.VMEM((2 * _NPW, _C), jnp.float32),
            pltpu.VMEM((2 * _NPW,), jnp.float32),
            pltpu.VMEM((2 * _NPW,), jnp.float32),
            pltpu.VMEM((_NPW,), jnp.float32),
            pltpu.VMEM((_NPW,), jnp.float32),
            pltpu.VMEM((_NPW,), jnp.float32),
            pltpu.VMEM((_NPW,), jnp.float32),
            pltpu.VMEM((_SROWS, 128), jnp.int32),
            pltpu.VMEM((_SROWS, 128), jnp.int32),
            pltpu.VMEM((_SROWS, 128), jnp.float32),
            pltpu.VMEM((128,), jnp.float32),
            pltpu.VMEM((_NS * 16,), jnp.float32),
            pltpu.VMEM((_NW * 16,), jnp.float32),
            pltpu.VMEM((16,), jnp.float32),
            pltpu.SemaphoreType.REGULAR,
        ] + [pltpu.SemaphoreType.DMA] * 18,
    )
    def k(h_hbm, src_hbm, dst_hbm, sens_hbm, sidx_hbm,
          out_hbm, zf_hbm, agg_hbm, red_hbm, deg_hbm, delta_hbm,
          zf_sh, agg_sh, deg_sh, delta_sh, red_sh,
          src_v, dst_v, gbufs_r, h_pl, z_pl, p_pl, rows_a, rows_b,
          zfstage, zero2d, stage640, zero1d, dinv_v, delta_v, dega_v,
          degb_v, sidx_v, sval_v, vals_v, ones_v, red_v, redg_v, vec16,
          xsem, *sems):
        gbufs = tuple(gbufs_r.at[b] for b in range(8))
        gsems = sems[:8]
        ssems = sems[8:]
        cid = lax.axis_index("c")
        sid = lax.axis_index("s")
        wid = cid * _NS + sid
        nbase = wid * _NPW
        sbase = sid * 2 * _NPW
        iota = lax.iota(jnp.int32, 16)

        def gbar():
            plsc.subcore_barrier()
            pl.semaphore_signal(xsem, 1, core_index=1 - cid)
            pl.semaphore_wait(xsem, 1)

        pltpu.sync_copy(src_hbm.at[pl.ds(wid * _ERW, _ERW)], src_v)
        pltpu.sync_copy(dst_hbm.at[pl.ds(wid * _ERW, _ERW)], dst_v)

        @pl.loop(0, 2 * _GPW)
        def _(g):
            zv = jnp.zeros((16,), jnp.float32)
            zero1d[pl.ds(g * 16, 16)] = zv
            ridx = g * 16 + iota
            for c in range(_C):
                plsc.store_scatter(zero2d, [ridx, _cidx(c)], zv)

        @pl.loop(0, 8)
        def _(g):
            ones_v[pl.ds(g * 16, 16)] = jnp.ones((16,), jnp.float32)

        pltpu.sync_copy(zero1d, deg_sh.at[pl.ds(sbase, 2 * _NPW)])
        pltpu.sync_copy(zero1d, delta_sh.at[pl.ds(sbase, 2 * _NPW)])
        plsc.subcore_barrier()

        @pl.loop(0, _ERW)
        def _(j):
            pltpu.sync_copy(ones_v, deg_sh.at[dst_v.at[j]], add=True)

        @pl.when(cid == 0)
        def _():
            pltpu.sync_copy(sidx_hbm.at[pl.ds(sid * _SROWS, _SROWS)], sidx_v)
            for r in range(_SROWS):
                pltpu.sync_copy(sens_hbm.at[sidx_v.at[r]], sval_v.at[r])
            acc = jnp.zeros((16,), jnp.float32)
            for r in range(_SROWS):
                for g in range(8):
                    acc = acc + sval_v[r, pl.ds(g * 16, 16)].astype(jnp.float32)
            vec16[...] = jnp.where(iota == 0, jnp.sum(acc), 0.0)
            pltpu.sync_copy(vec16, red_sh.at[pl.ds(sid * 16, 16)])
            plsc.subcore_barrier()
            pltpu.sync_copy(red_sh, red_v)
            tot = red_v[pl.ds(0, 16)]
            for i in range(1, _NS):
                tot = tot + red_v[pl.ds(i * 16, 16)]
            n1b = jnp.full((16,), tot[0])
            r1v = 1.0 / jnp.maximum(n1b, 1.0)
            r0v = -1.0 / jnp.maximum(float(_NTRAIN) - n1b, 1.0)
            for r in range(_SROWS):
                for g in range(8):
                    sl = pl.ds(g * 16, 16)
                    sv = sval_v[r, sl]
                    ix = sidx_v[r, sl]
                    val = jnp.where(sv > 0, r1v, r0v)
                    vals_v[r, sl] = jnp.where(ix < _N, val, 0.0)
            for r in range(_SROWS):
                pltpu.sync_copy(vals_v.at[r], delta_sh.at[sidx_v.at[r]],
                                add=True)
            plsc.subcore_barrier()
            pltpu.sync_copy(delta_sh.at[pl.ds(sbase, 2 * _NPW)], stage640)
            pltpu.sync_copy(stage640, delta_hbm.at[pl.ds(sbase, 2 * _NPW)])

        plsc.subcore_barrier()
        pltpu.sync_copy(deg_sh.at[pl.ds(sbase, 2 * _NPW)], stage640)
        pltpu.sync_copy(stage640,
                        deg_hbm.at[pl.ds(cid * _NPAD + sbase, 2 * _NPW)])
        gbar()

        pltpu.sync_copy(deg_hbm.at[pl.ds(nbase, _NPW)], dega_v)
        pltpu.sync_copy(deg_hbm.at[pl.ds(_NPAD + nbase, _NPW)], degb_v)
        pltpu.sync_copy(delta_hbm.at[pl.ds(nbase, _NPW)], delta_v)
        pltpu.sync_copy(h_hbm.at[pl.ds(nbase, _NPW)], rows_a)

        @pl.loop(0, _GPW)
        def _(g):
            sl = pl.ds(g * 16, 16)
            d = jnp.maximum(dega_v[sl] + degb_v[sl], 1.0)
            db = plsc.bitcast(d, jnp.int32)
            y = plsc.bitcast(jnp.int32(0x5F3759DF) - (db >> 1), jnp.float32)
            for _i in range(3):
                y = y * (1.5 - 0.5 * d * y * y)
            dinv_v[sl] = y
            ridx = g * 16 + iota
            for c in range(_C):
                hv = plsc.load_gather(rows_a, [ridx, _cidx(c)])
                h_pl[c, sl] = hv
                z_pl[c, sl] = hv

        def step(_k, u):
            def ga(g, accs):
                sl = pl.ds(g * 16, 16)
                zs = [z_pl[c, sl] for c in range(_C)]
                m = zs[0]
                for c in range(1, _C):
                    m = jnp.maximum(m, zs[c])
                es = [jnp.exp(zs[c] - m) for c in range(_C)]
                s = es[0]
                for c in range(1, _C):
                    s = s + es[c]
                rinv = 1.0 / s
                dv = delta_v[sl]
                out = []
                for c in range(_C):
                    p = es[c] * rinv
                    p_pl[c, sl] = p
                    out.append(accs[c] + dv * p)
                return tuple(out)

            accs = lax.fori_loop(
                0, _GPW, ga,
                tuple(jnp.zeros((16,), jnp.float32) for _ in range(_C)))
            ap = jnp.zeros((16,), jnp.float32)
            for c in range(_C):
                ap = jnp.where(iota == c, jnp.sum(accs[c]), ap)
            vec16[...] = ap
            pltpu.sync_copy(vec16, red_hbm.at[pl.ds(wid * 16, 16)])
            gbar()
            pltpu.make_async_copy(zero2d, agg_sh.at[pl.ds(sbase, 2 * _NPW)],
                                  sems[16]).start()
            pltpu.sync_copy(red_hbm, redg_v)
            a = redg_v[pl.ds(0, 16)]
            for i in range(1, _NW):
                a = a + redg_v[pl.ds(i * 16, 16)]
            u = jnp.clip(u + _BETA * a, -_LAMBDA2, _LAMBDA2)
            us = [jnp.full((16,), u[c]) for c in range(_C)]

            def gb(g, carry):
                sl = pl.ds(g * 16, 16)
                ps = [p_pl[c, sl] for c in range(_C)]
                pu = ps[0] * us[0]
                for c in range(1, _C):
                    pu = pu + ps[c] * us[c]
                dv = delta_v[sl] * _GAMMA
                divv = dinv_v[sl]
                ridx = g * 16 + iota
                for c in range(_C):
                    zf = (z_pl[c, sl] - dv * (ps[c] * (us[c] - pu))) * divv
                    plsc.store_scatter(rows_a, [ridx, _cidx(c)], zf)
                return carry

            lax.fori_loop(0, _GPW, gb, 0)
            zfw = pltpu.make_async_copy(
                rows_a, zf_hbm.at[pl.ds(nbase, _NPW)], sems[17])
            zfw.start()
            pltpu.sync_copy(rows_a, zf_sh.at[pl.ds(nbase, _NPW)])
            zfw.wait()
            gbar()

            pbase = (1 - cid) * (_NS * _NPW) + sid * _NPW
            pltpu.sync_copy(zf_hbm.at[pl.ds(pbase, _NPW)], rows_b)
            pltpu.sync_copy(rows_b, zf_sh.at[pl.ds(pbase, _NPW)])
            pltpu.make_async_copy(zero2d, agg_sh.at[pl.ds(sbase, 2 * _NPW)],
                                  sems[16]).wait()
            plsc.subcore_barrier()

            def _gath(j, b):
                return pltpu.make_async_copy(
                    zf_sh.at[src_v.at[j]], gbufs[b], gsems[b])

            def _scat(j, b):
                return pltpu.make_async_copy(
                    gbufs[b], agg_sh.at[dst_v.at[j]], ssems[b])

            for b0 in range(4):
                _gath(b0, b0).start()
            for j0 in range(4):
                _gath(j0, j0).wait()
                _scat(j0, j0).start(add=True)
                _gath(j0 + 4, j0 + 4).start()

            @pl.loop(0, (_ERW - 8) // 8)
            def _(i):
                j = 8 * i + 4
                for t in range(8):
                    b = (4 + t) % 8
                    _gath(j + t, b).wait()
                    _scat(j + t, b).start(add=True)
                    b2 = (b + 4) % 8
                    _scat(j + t - 4, b2).wait()
                    _gath(j + t + 4, b2).start()

            for t in range(4):
                j0, b0 = _ERW - 4 + t, (4 + t) % 8
                _gath(j0, b0).wait()
                _scat(j0, b0).start(add=True)
                _scat(j0 - 4, (b0 + 4) % 8).wait()
            for t in range(4):
                _scat(_ERW - 4 + t, (4 + t) % 8).wait()

            plsc.subcore_barrier()

            pltpu.sync_copy(agg_sh.at[pl.ds(sbase, 2 * _NPW)], zfstage)
            pltpu.sync_copy(
                zfstage, agg_hbm.at[pl.ds(cid * _NPAD + sbase, 2 * _NPW)])
            gbar()

            aga = pltpu.make_async_copy(
                agg_hbm.at[pl.ds((1 - cid) * _NPAD + nbase, _NPW)], rows_a,
                sems[17])
            aga.start()
            pltpu.sync_copy(agg_sh.at[pl.ds(nbase, _NPW)], rows_b)
            aga.wait()

            def gc(g, carry):
                sl = pl.ds(g * 16, 16)
                divv = dinv_v[sl]
                ridx = g * 16 + iota
                for c in range(_C):
                    av = (plsc.load_gather(rows_a, [ridx, _cidx(c)])
                          + plsc.load_gather(rows_b, [ridx, _cidx(c)]))
                    z_pl[c, sl] = (_GAMMA * h_pl[c, sl]
                                   + (1.0 - _GAMMA) * divv * av)
                return carry

            lax.fori_loop(0, _GPW, gc, 0)
            return u

        lax.fori_loop(0, _K_PROP, step, jnp.zeros((16,), jnp.float32))

        def go(g, carry):
            sl = pl.ds(g * 16, 16)
            ridx = g * 16 + iota
            for c in range(_C):
                plsc.store_scatter(rows_a, [ridx, _cidx(c)], z_pl[c, sl])
            return carry

        lax.fori_loop(0, _GPW, go, 0)
        pltpu.sync_copy(rows_a, out_hbm.at[pl.ds(nbase, _NPW)])

    return k(h, srcp, dstp, sensp, sidxp)[0]


def kernel(x, edge_index, sensitive_attr, idx_sens_train, W1, b1, Wh, bh, Wl, bl):
    xp = jnp.pad(x, ((0, _NPAD - _N), (0, 0)))
    h = _mlp(xp, W1, b1, Wh, bh, Wl, bl)

    n_epad = _EPAD - _E
    pad_idx = (jnp.arange(n_epad, dtype=jnp.int32) % (_NPAD - _N)) + _N
    srcp = jnp.concatenate([edge_index[0], pad_idx]).reshape(_EPAD // 128, 128)
    dstp = jnp.concatenate([edge_index[1], pad_idx]).reshape(_EPAD // 128, 128)
    sensp = jnp.pad(sensitive_attr, (0, _NPAD - _N))
    sidxp = jnp.concatenate(
        [idx_sens_train,
         jnp.full((_SPAD - _NTRAIN,), _N, jnp.int32)]).reshape(_SPAD // 128, 128)

    z = _prop(h, srcp, dstp, sensp, sidxp)
    return z[:_N]
